# baseline (speedup 1.0000x reference)
import functools

import jax
import jax.numpy as jnp
from jax import lax
from jax.experimental import pallas as pl
from jax.experimental.pallas import tpu as pltpu
from jax.experimental.pallas import tpu_sc as plsc

N = 10000
D = 128
E = 320000
K = 10
ALPHA = 0.1
EPS = 1e-5

NC = 2
NS = 16
B = 128
NB = 80
NBG = NB + 2
EP = NC * NS * NB * B
TRASH = N
RPT = 632
NP = NS * RPT

_mesh = plsc.VectorSubcoreMesh(core_axis_name="c", subcore_axis_name="s")


@functools.partial(
    pl.kernel,
    out_type=jax.ShapeDtypeStruct((NC, NP, D), jnp.float32),
    mesh=_mesh,
    scratch_types=[
        pltpu.VMEM_SHARED((NP, D), jnp.float32),
        pltpu.VMEM((NBG, B), jnp.int32),
        pltpu.VMEM((4, B), jnp.int32),
        pltpu.VMEM((4, B), jnp.int32),
        pltpu.VMEM((B, D), jnp.float32),
        pltpu.VMEM((B, D), jnp.float32),
        pltpu.SemaphoreType.DMA,
        pltpu.SemaphoreType.DMA,
    ],
)
def _sc_propagate(g_hbm, packed_hbm, out_hbm,
                  acc_sh, pk_v, sidx_v, didx_v, rows0_v, rows1_v, sem0, sem1):
    c = lax.axis_index("c")
    s = lax.axis_index("s")
    tid = c * NS + s
    base = s * RPT
    pltpu.sync_copy(g_hbm.at[pl.ds(base, RPT)], acc_sh.at[pl.ds(base, RPT)])
    pltpu.sync_copy(packed_hbm.at[tid], pk_v)
    plsc.subcore_barrier()

    sh = jnp.full((16,), 14, jnp.int32)
    msk = jnp.full((16,), 16383, jnp.int32)

    def unpack(j, slot):
        for b in range(B // 16):
            v = pk_v[j, pl.ds(b * 16, 16)]
            sidx_v[slot, pl.ds(b * 16, 16)] = lax.shift_right_logical(v, sh)
            didx_v[slot, pl.ds(b * 16, 16)] = lax.bitwise_and(v, msk)

    rows = (rows0_v, rows1_v)
    sems = (sem0, sem1)

    for r in (0, 1):
        unpack(r, r)
        pltpu.async_copy(g_hbm.at[sidx_v.at[r]], rows[r], sems[r])

    def body(i, carry):
        j4 = 4 * i
        for r in range(4):
            rb = r % 2
            pltpu.make_async_copy(g_hbm.at[pl.ds(0, B)], rows[rb], sems[rb]).wait()
            unpack(j4 + r + 2, (r + 2) % 4)
            pltpu.sync_copy(rows[rb], acc_sh.at[didx_v.at[r]], add=True)
            pltpu.async_copy(g_hbm.at[sidx_v.at[(r + 2) % 4]], rows[rb], sems[rb])
        return carry

    lax.fori_loop(0, NB // 4, body, 0)
    pltpu.make_async_copy(g_hbm.at[pl.ds(0, B)], rows0_v, sem0).wait()
    pltpu.make_async_copy(g_hbm.at[pl.ds(0, B)], rows1_v, sem1).wait()
    plsc.subcore_barrier()
    pltpu.sync_copy(acc_sh.at[pl.ds(base, RPT)], out_hbm.at[c, pl.ds(base, RPT)])


_BLK = 1000


def _mlp_body(x_ref, w1_ref, b1_ref, lng_ref, lnb_ref, wo_ref, bo_ref, pd_ref,
              h0_ref, g0_ref, ca_ref, cb_ref):
    x = x_ref[...]
    h = jnp.dot(x, w1_ref[...], preferred_element_type=jnp.float32) + b1_ref[...]
    mu = jnp.mean(h, axis=-1, keepdims=True)
    var = jnp.mean((h - mu) ** 2, axis=-1, keepdims=True)
    h = (h - mu) * lax.rsqrt(var + EPS) * lng_ref[...] + lnb_ref[...]
    h = jnp.maximum(h, 0.0)
    h0 = jnp.dot(h, wo_ref[...], preferred_element_type=jnp.float32) + bo_ref[...]
    pd = pd_ref[...]
    deg = pd[0, :, 0:1] + pd[1, :, 0:1] - 1.0
    dinv = lax.rsqrt(deg)
    h0_ref[...] = h0
    g0_ref[...] = dinv * h0
    ca_ref[...] = (1.0 - ALPHA) * dinv * dinv
    cb_ref[...] = (1.0 - ALPHA) * dinv


_mlp_call = pl.pallas_call(
    _mlp_body,
    grid=(N // _BLK,),
    in_specs=[
        pl.BlockSpec((_BLK, D), lambda i: (i, 0)),
        pl.BlockSpec((D, D), lambda i: (0, 0)),
        pl.BlockSpec((1, D), lambda i: (0, 0)),
        pl.BlockSpec((1, D), lambda i: (0, 0)),
        pl.BlockSpec((1, D), lambda i: (0, 0)),
        pl.BlockSpec((D, D), lambda i: (0, 0)),
        pl.BlockSpec((1, D), lambda i: (0, 0)),
        pl.BlockSpec((NC, _BLK, D), lambda i: (0, i, 0)),
    ],
    out_specs=[
        pl.BlockSpec((_BLK, D), lambda i: (i, 0)),
        pl.BlockSpec((_BLK, D), lambda i: (i, 0)),
        pl.BlockSpec((_BLK, 1), lambda i: (i, 0)),
        pl.BlockSpec((_BLK, 1), lambda i: (i, 0)),
    ],
    out_shape=[
        jax.ShapeDtypeStruct((N, D), jnp.float32),
        jax.ShapeDtypeStruct((N, D), jnp.float32),
        jax.ShapeDtypeStruct((N, 1), jnp.float32),
        jax.ShapeDtypeStruct((N, 1), jnp.float32),
    ],
)


def _combine_body(p_ref, g_ref, c_ref, dt_ref, o_ref):
    p = p_ref[...]
    o_ref[...] = c_ref[...] * (p[0] + p[1] - g_ref[...]) + ALPHA * dt_ref[...]


_combine_call = pl.pallas_call(
    _combine_body,
    grid=(NS,),
    in_specs=[
        pl.BlockSpec((NC, RPT, D), lambda i: (0, i, 0)),
        pl.BlockSpec((RPT, D), lambda i: (i, 0)),
        pl.BlockSpec((RPT, 1), lambda i: (i, 0)),
        pl.BlockSpec((RPT, D), lambda i: (i, 0)),
    ],
    out_specs=pl.BlockSpec((RPT, D), lambda i: (i, 0)),
    out_shape=jax.ShapeDtypeStruct((NP, D), jnp.float32),
)


def kernel(x, edge_index, W1, b1, ln_g, ln_b, Wout, bout):
    src = edge_index[0].astype(jnp.int32)
    dst = edge_index[1].astype(jnp.int32)
    pad = EP - E
    srcp = jnp.concatenate([src, jnp.zeros((pad,), jnp.int32)])
    dstp = jnp.concatenate([dst, jnp.full((pad,), TRASH, jnp.int32)])
    packed = ((srcp << 14) | dstp).reshape(NC * NS, NB, B)
    packed = jnp.concatenate(
        [packed, jnp.full((NC * NS, NBG - NB, B), TRASH, jnp.int32)], axis=1)

    ones_col = jnp.concatenate(
        [jnp.ones((N, D), jnp.float32), jnp.zeros((NP - N, D), jnp.float32)])
    pdeg = _sc_propagate(ones_col, packed)

    h0, g0, cA, cB = _mlp_call(
        x, W1, b1.reshape(1, D), ln_g.reshape(1, D), ln_b.reshape(1, D),
        Wout, bout.reshape(1, D), pdeg)

    zpad = ((0, NP - N), (0, 0))
    h0 = jnp.pad(h0, zpad)
    g0 = jnp.pad(g0, zpad)
    cA = jnp.pad(cA, zpad)
    cB = jnp.pad(cB, zpad)

    g = g0
    for k in range(K):
        p = _sc_propagate(g, packed)
        if k < K - 1:
            g = _combine_call(p, g, cA, g0)
        else:
            g = _combine_call(p, g, cB, h0)
    return g[:N]


# speedup vs baseline: 2.2787x; 2.2787x over previous
import functools

import jax
import jax.numpy as jnp
from jax import lax
from jax.experimental import pallas as pl
from jax.experimental.pallas import tpu as pltpu
from jax.experimental.pallas import tpu_sc as plsc

N = 10000
D = 128
E = 320000
K = 10
ALPHA = 0.1
EPS = 1e-5

NC = 2
NS = 16
B = 128
NB = 79
EP = NC * NS * NB * B
TRASH = N
RPT = 632
NP = NS * RPT

_mesh = plsc.VectorSubcoreMesh(core_axis_name="c", subcore_axis_name="s")


@functools.partial(
    pl.kernel,
    out_type=jax.ShapeDtypeStruct((NC, NP, D), jnp.float32),
    mesh=_mesh,
    scratch_types=[
        pltpu.VMEM_SHARED((NP, D), jnp.float32),
        pltpu.VMEM((NB, B), jnp.int32),
        pltpu.VMEM((NB, B), jnp.int32),
        pltpu.VMEM((B, D), jnp.float32),
        pltpu.SemaphoreType.DMA,
    ],
)
def _sc_propagate(g_hbm, srcs_hbm, dsts_hbm, out_hbm,
                  acc_sh, src_v, dst_v, rows_v, sem):
    c = lax.axis_index("c")
    s = lax.axis_index("s")
    tid = c * NS + s
    base = s * RPT
    pltpu.sync_copy(g_hbm.at[pl.ds(base, RPT)], acc_sh.at[pl.ds(base, RPT)])
    pltpu.sync_copy(srcs_hbm.at[tid], src_v)
    pltpu.sync_copy(dsts_hbm.at[tid], dst_v)
    plsc.subcore_barrier()

    def body(j, carry):
        pltpu.async_copy(g_hbm.at[src_v.at[j]], rows_v, sem).wait()
        pltpu.sync_copy(rows_v, acc_sh.at[dst_v.at[j]], add=True)
        return carry

    lax.fori_loop(0, NB, body, 0)
    plsc.subcore_barrier()
    pltpu.sync_copy(acc_sh.at[pl.ds(base, RPT)], out_hbm.at[c, pl.ds(base, RPT)])


_BLK = 1000


def _mlp_body(x_ref, w1_ref, b1_ref, lng_ref, lnb_ref, wo_ref, bo_ref, pd_ref,
              h0_ref, g0_ref, ca_ref, cb_ref):
    x = x_ref[...]
    h = jnp.dot(x, w1_ref[...], preferred_element_type=jnp.float32) + b1_ref[...]
    mu = jnp.mean(h, axis=-1, keepdims=True)
    var = jnp.mean((h - mu) ** 2, axis=-1, keepdims=True)
    h = (h - mu) * lax.rsqrt(var + EPS) * lng_ref[...] + lnb_ref[...]
    h = jnp.maximum(h, 0.0)
    h0 = jnp.dot(h, wo_ref[...], preferred_element_type=jnp.float32) + bo_ref[...]
    pd = pd_ref[...]
    deg = pd[0, :, 0:1] + pd[1, :, 0:1] - 1.0
    dinv = lax.rsqrt(deg)
    h0_ref[...] = h0
    g0_ref[...] = dinv * h0
    ca_ref[...] = (1.0 - ALPHA) * dinv * dinv
    cb_ref[...] = (1.0 - ALPHA) * dinv


_mlp_call = pl.pallas_call(
    _mlp_body,
    grid=(N // _BLK,),
    in_specs=[
        pl.BlockSpec((_BLK, D), lambda i: (i, 0)),
        pl.BlockSpec((D, D), lambda i: (0, 0)),
        pl.BlockSpec((1, D), lambda i: (0, 0)),
        pl.BlockSpec((1, D), lambda i: (0, 0)),
        pl.BlockSpec((1, D), lambda i: (0, 0)),
        pl.BlockSpec((D, D), lambda i: (0, 0)),
        pl.BlockSpec((1, D), lambda i: (0, 0)),
        pl.BlockSpec((NC, _BLK, D), lambda i: (0, i, 0)),
    ],
    out_specs=[
        pl.BlockSpec((_BLK, D), lambda i: (i, 0)),
        pl.BlockSpec((_BLK, D), lambda i: (i, 0)),
        pl.BlockSpec((_BLK, 1), lambda i: (i, 0)),
        pl.BlockSpec((_BLK, 1), lambda i: (i, 0)),
    ],
    out_shape=[
        jax.ShapeDtypeStruct((N, D), jnp.float32),
        jax.ShapeDtypeStruct((N, D), jnp.float32),
        jax.ShapeDtypeStruct((N, 1), jnp.float32),
        jax.ShapeDtypeStruct((N, 1), jnp.float32),
    ],
)


def _combine_body(p_ref, g_ref, c_ref, dt_ref, o_ref):
    p = p_ref[...]
    o_ref[...] = c_ref[...] * (p[0] + p[1] - g_ref[...]) + ALPHA * dt_ref[...]


_combine_call = pl.pallas_call(
    _combine_body,
    grid=(NS,),
    in_specs=[
        pl.BlockSpec((NC, RPT, D), lambda i: (0, i, 0)),
        pl.BlockSpec((RPT, D), lambda i: (i, 0)),
        pl.BlockSpec((RPT, 1), lambda i: (i, 0)),
        pl.BlockSpec((RPT, D), lambda i: (i, 0)),
    ],
    out_specs=pl.BlockSpec((RPT, D), lambda i: (i, 0)),
    out_shape=jax.ShapeDtypeStruct((NP, D), jnp.float32),
)


def kernel(x, edge_index, W1, b1, ln_g, ln_b, Wout, bout):
    src = edge_index[0].astype(jnp.int32)
    dst = edge_index[1].astype(jnp.int32)
    pad = EP - E
    srcs = jnp.concatenate([src, jnp.zeros((pad,), jnp.int32)]).reshape(
        NC * NS, NB, B)
    dsts = jnp.concatenate([dst, jnp.full((pad,), TRASH, jnp.int32)]).reshape(
        NC * NS, NB, B)

    ones_col = jnp.concatenate(
        [jnp.ones((N, D), jnp.float32), jnp.zeros((NP - N, D), jnp.float32)])
    pdeg = _sc_propagate(ones_col, srcs, dsts)

    h0, g0, cA, cB = _mlp_call(
        x, W1, b1.reshape(1, D), ln_g.reshape(1, D), ln_b.reshape(1, D),
        Wout, bout.reshape(1, D), pdeg)

    zpad = ((0, NP - N), (0, 0))
    h0 = jnp.pad(h0, zpad)
    g0 = jnp.pad(g0, zpad)
    cA = jnp.pad(cA, zpad)
    cB = jnp.pad(cB, zpad)

    g = g0
    for k in range(K):
        p = _sc_propagate(g, srcs, dsts)
        if k < K - 1:
            g = _combine_call(p, g, cA, g0)
        else:
            g = _combine_call(p, g, cB, h0)
    return g[:N]


# speedup vs baseline: 2.4380x; 1.0699x over previous
import functools

import jax
import jax.numpy as jnp
from jax import lax
from jax.experimental import pallas as pl
from jax.experimental.pallas import tpu as pltpu
from jax.experimental.pallas import tpu_sc as plsc

N = 10000
D = 128
E = 320000
K = 10
ALPHA = 0.1
EPS = 1e-5

NC = 2
NS = 16
B = 128
NB = 79
EP = NC * NS * NB * B
TRASH = N
RPT = 632
NP = NS * RPT

_mesh = plsc.VectorSubcoreMesh(core_axis_name="c", subcore_axis_name="s")


@functools.partial(
    pl.kernel,
    out_type=jax.ShapeDtypeStruct((NC, NP, D), jnp.float32),
    mesh=_mesh,
    scratch_types=[
        pltpu.VMEM_SHARED((NP, D), jnp.float32),
        pltpu.VMEM((NB, B), jnp.int32),
        pltpu.VMEM((B, D), jnp.float32),
    ],
)
def _sc_degree(dsts_hbm, zeros_hbm, ones_hbm, out_hbm, deg_sh, dst_v, ones_v):
    c = lax.axis_index("c")
    s = lax.axis_index("s")
    tid = c * NS + s
    base = s * RPT
    pltpu.sync_copy(zeros_hbm.at[pl.ds(base, RPT)], deg_sh.at[pl.ds(base, RPT)])
    pltpu.sync_copy(ones_hbm, ones_v)
    pltpu.sync_copy(dsts_hbm.at[tid], dst_v)
    plsc.subcore_barrier()

    def body(j, carry):
        pltpu.sync_copy(ones_v, deg_sh.at[dst_v.at[j]], add=True)
        return carry

    lax.fori_loop(0, NB, body, 0)
    plsc.subcore_barrier()
    pltpu.sync_copy(deg_sh.at[pl.ds(base, RPT)], out_hbm.at[c, pl.ds(base, RPT)])


@functools.partial(
    pl.kernel,
    out_type=jax.ShapeDtypeStruct((NC, NP, D), jnp.float32),
    mesh=_mesh,
    scratch_types=[
        pltpu.VMEM_SHARED((NP, D), jnp.float32),
        pltpu.VMEM((NB, B), jnp.int32),
        pltpu.VMEM((NB, B), jnp.int32),
        pltpu.VMEM((B, D), jnp.float32),
        pltpu.SemaphoreType.DMA,
    ],
)
def _sc_propagate(g_hbm, srcs_hbm, dsts_hbm, out_hbm,
                  acc_sh, src_v, dst_v, rows_v, sem):
    c = lax.axis_index("c")
    s = lax.axis_index("s")
    tid = c * NS + s
    base = s * RPT
    pltpu.sync_copy(g_hbm.at[pl.ds(base, RPT)], acc_sh.at[pl.ds(base, RPT)])
    pltpu.sync_copy(srcs_hbm.at[tid], src_v)
    pltpu.sync_copy(dsts_hbm.at[tid], dst_v)
    plsc.subcore_barrier()

    def body(j, carry):
        pltpu.async_copy(g_hbm.at[src_v.at[j]], rows_v, sem).wait()
        pltpu.sync_copy(rows_v, acc_sh.at[dst_v.at[j]], add=True)
        return carry

    lax.fori_loop(0, NB, body, 0)
    plsc.subcore_barrier()
    pltpu.sync_copy(acc_sh.at[pl.ds(base, RPT)], out_hbm.at[c, pl.ds(base, RPT)])


_BLK = 1000


def _mlp_body(x_ref, w1_ref, b1_ref, lng_ref, lnb_ref, wo_ref, bo_ref, pd_ref,
              h0_ref, g0_ref, ca_ref, cb_ref):
    x = x_ref[...]
    h = jnp.dot(x, w1_ref[...], preferred_element_type=jnp.float32) + b1_ref[...]
    mu = jnp.mean(h, axis=-1, keepdims=True)
    var = jnp.mean((h - mu) ** 2, axis=-1, keepdims=True)
    h = (h - mu) * lax.rsqrt(var + EPS) * lng_ref[...] + lnb_ref[...]
    h = jnp.maximum(h, 0.0)
    h0 = jnp.dot(h, wo_ref[...], preferred_element_type=jnp.float32) + bo_ref[...]
    pd = pd_ref[...]
    deg = pd[0, :, 0:1] + pd[1, :, 0:1] + 1.0
    dinv = lax.rsqrt(deg)
    h0_ref[...] = h0
    g0_ref[...] = dinv * h0
    ca_ref[...] = (1.0 - ALPHA) * dinv * dinv
    cb_ref[...] = (1.0 - ALPHA) * dinv


_mlp_call = pl.pallas_call(
    _mlp_body,
    grid=(N // _BLK,),
    in_specs=[
        pl.BlockSpec((_BLK, D), lambda i: (i, 0)),
        pl.BlockSpec((D, D), lambda i: (0, 0)),
        pl.BlockSpec((1, D), lambda i: (0, 0)),
        pl.BlockSpec((1, D), lambda i: (0, 0)),
        pl.BlockSpec((1, D), lambda i: (0, 0)),
        pl.BlockSpec((D, D), lambda i: (0, 0)),
        pl.BlockSpec((1, D), lambda i: (0, 0)),
        pl.BlockSpec((NC, _BLK, D), lambda i: (0, i, 0)),
    ],
    out_specs=[
        pl.BlockSpec((_BLK, D), lambda i: (i, 0)),
        pl.BlockSpec((_BLK, D), lambda i: (i, 0)),
        pl.BlockSpec((_BLK, 1), lambda i: (i, 0)),
        pl.BlockSpec((_BLK, 1), lambda i: (i, 0)),
    ],
    out_shape=[
        jax.ShapeDtypeStruct((N, D), jnp.float32),
        jax.ShapeDtypeStruct((N, D), jnp.float32),
        jax.ShapeDtypeStruct((N, 1), jnp.float32),
        jax.ShapeDtypeStruct((N, 1), jnp.float32),
    ],
)


def _combine_body(p_ref, g_ref, c_ref, dt_ref, o_ref):
    p = p_ref[...]
    o_ref[...] = c_ref[...] * (p[0] + p[1] - g_ref[...]) + ALPHA * dt_ref[...]


_combine_call = pl.pallas_call(
    _combine_body,
    grid=(NS,),
    in_specs=[
        pl.BlockSpec((NC, RPT, D), lambda i: (0, i, 0)),
        pl.BlockSpec((RPT, D), lambda i: (i, 0)),
        pl.BlockSpec((RPT, 1), lambda i: (i, 0)),
        pl.BlockSpec((RPT, D), lambda i: (i, 0)),
    ],
    out_specs=pl.BlockSpec((RPT, D), lambda i: (i, 0)),
    out_shape=jax.ShapeDtypeStruct((NP, D), jnp.float32),
)


def kernel(x, edge_index, W1, b1, ln_g, ln_b, Wout, bout):
    src = edge_index[0].astype(jnp.int32)
    dst = edge_index[1].astype(jnp.int32)
    pad = EP - E
    srcs = jnp.concatenate([src, jnp.zeros((pad,), jnp.int32)]).reshape(
        NC * NS, NB, B)
    dsts = jnp.concatenate([dst, jnp.full((pad,), TRASH, jnp.int32)]).reshape(
        NC * NS, NB, B)

    zerosd = jnp.zeros((NP, D), jnp.float32)
    onesd = jnp.ones((B, D), jnp.float32)
    pdeg = _sc_degree(dsts, zerosd, onesd)

    h0, g0, cA, cB = _mlp_call(
        x, W1, b1.reshape(1, D), ln_g.reshape(1, D), ln_b.reshape(1, D),
        Wout, bout.reshape(1, D), pdeg)

    zpad = ((0, NP - N), (0, 0))
    h0 = jnp.pad(h0, zpad)
    g0 = jnp.pad(g0, zpad)
    cA = jnp.pad(cA, zpad)
    cB = jnp.pad(cB, zpad)

    g = g0
    for k in range(K):
        p = _sc_propagate(g, srcs, dsts)
        if k < K - 1:
            g = _combine_call(p, g, cA, g0)
        else:
            g = _combine_call(p, g, cB, h0)
    return g[:N]
